# Initial kernel scaffold; baseline (speedup 1.0000x reference)
import functools
import math

import jax
import jax.numpy as jnp
from jax import lax
from jax.experimental import pallas as pl
from jax.experimental.pallas import tpu as pltpu
from jax.experimental.pallas import tpu_sc as plsc

VOCAB = 1_000_000
EMB = 32
BATCH = 4096
SEQ = 200
N = BATCH * SEQ

_info = plsc.get_sparse_core_info()
NC = _info.num_cores
NS = _info.num_subcores
NW = NC * NS
PER_W = N // NW
CHUNK = 1600
NCHUNK = PER_W // CHUNK
SCALE = math.sqrt(EMB)

_mesh = plsc.VectorSubcoreMesh(core_axis_name="c", subcore_axis_name="s")


@functools.partial(
    pl.kernel,
    out_type=jax.ShapeDtypeStruct((N, EMB), jnp.float32),
    mesh=_mesh,
    scratch_types=[
        pltpu.VMEM((CHUNK,), jnp.int32),
        pltpu.VMEM((CHUNK, EMB), jnp.float32),
        pltpu.SemaphoreType.DMA,
    ],
)
def _embed_sc(tok_hbm, table_hbm, out_hbm, idx_v, rows_v, sem):
    wid = lax.axis_index("s") * NC + lax.axis_index("c")
    base = wid * PER_W

    def chunk_body(c, carry):
        off = base + c * CHUNK
        pltpu.sync_copy(tok_hbm.at[pl.ds(off, CHUNK)], idx_v)
        pltpu.async_copy(table_hbm.at[idx_v], rows_v, sem).wait()

        def scale_row(j, carry2):
            lo = rows_v[j, pl.ds(0, 16)] * SCALE
            hi = rows_v[j, pl.ds(16, 16)] * SCALE
            rows_v[j, pl.ds(0, 16)] = lo
            rows_v[j, pl.ds(16, 16)] = hi
            return carry2

        lax.fori_loop(0, CHUNK, scale_row, 0, unroll=4)
        pltpu.sync_copy(rows_v, out_hbm.at[pl.ds(off, CHUNK)])
        return carry

    lax.fori_loop(0, NCHUNK, chunk_body, 0)


def kernel(tokens, embedding_weight):
    flat = tokens.reshape(N)
    out = _embed_sc(flat, embedding_weight)
    return out.reshape(BATCH, SEQ, EMB)


# speedup vs baseline: 1.4169x; 1.4169x over previous
import functools
import math

import jax
import jax.numpy as jnp
from jax import lax
from jax.experimental import pallas as pl
from jax.experimental.pallas import tpu as pltpu
from jax.experimental.pallas import tpu_sc as plsc

VOCAB = 1_000_000
EMB = 32
BATCH = 4096
SEQ = 200
N = BATCH * SEQ

_info = plsc.get_sparse_core_info()
NC = _info.num_cores
NS = _info.num_subcores
NW = NC * NS
PER_W = N // NW
CHUNK = 1600
NCHUNK = PER_W // CHUNK
SCALE = math.sqrt(EMB)

_mesh = plsc.VectorSubcoreMesh(core_axis_name="c", subcore_axis_name="s")


@functools.partial(
    pl.kernel,
    out_type=jax.ShapeDtypeStruct((N, EMB), jnp.float32),
    mesh=_mesh,
    scratch_types=[
        pltpu.VMEM((CHUNK,), jnp.int32),
        pltpu.VMEM((CHUNK, EMB), jnp.float32),
        pltpu.SemaphoreType.DMA,
    ],
    compiler_params=pltpu.CompilerParams(use_tc_tiling_on_sc=False),
)
def _embed_sc(tok_hbm, table_hbm, out_hbm, idx_v, rows_v, sem):
    wid = lax.axis_index("s") * NC + lax.axis_index("c")
    base = wid * PER_W

    def chunk_body(c, carry):
        off = base + c * CHUNK
        pltpu.sync_copy(tok_hbm.at[pl.ds(off, CHUNK)], idx_v)
        pltpu.async_copy(table_hbm.at[idx_v], rows_v, sem).wait()

        def scale_row(j, carry2):
            lo = rows_v[j, pl.ds(0, 16)] * SCALE
            hi = rows_v[j, pl.ds(16, 16)] * SCALE
            rows_v[j, pl.ds(0, 16)] = lo
            rows_v[j, pl.ds(16, 16)] = hi
            return carry2

        lax.fori_loop(0, CHUNK, scale_row, 0, unroll=4)
        pltpu.sync_copy(rows_v, out_hbm.at[pl.ds(off, CHUNK)])
        return carry

    lax.fori_loop(0, NCHUNK, chunk_body, 0)


def kernel(tokens, embedding_weight):
    flat = tokens.reshape(N)
    out = _embed_sc(flat, embedding_weight)
    return out.reshape(BATCH, SEQ, EMB)


# speedup vs baseline: 1.4770x; 1.0424x over previous
import functools
import math

import jax
import jax.numpy as jnp
from jax import lax
from jax.experimental import pallas as pl
from jax.experimental.pallas import tpu as pltpu
from jax.experimental.pallas import tpu_sc as plsc

VOCAB = 1_000_000
EMB = 32
BATCH = 4096
SEQ = 200
N = BATCH * SEQ

_info = plsc.get_sparse_core_info()
NC = _info.num_cores
NS = _info.num_subcores
NW = NC * NS
PER_W = N // NW
CHUNK = 1600
NCHUNK = PER_W // CHUNK
SCALE = math.sqrt(EMB)

_mesh = plsc.VectorSubcoreMesh(core_axis_name="c", subcore_axis_name="s")


@functools.partial(
    pl.kernel,
    out_type=jax.ShapeDtypeStruct((N, EMB), jnp.float32),
    mesh=_mesh,
    scratch_types=[
        pltpu.VMEM((NCHUNK, CHUNK), jnp.int32),
        pltpu.VMEM((CHUNK, EMB), jnp.float32),
        pltpu.VMEM((CHUNK, EMB), jnp.float32),
        pltpu.SemaphoreType.DMA,
        pltpu.SemaphoreType.DMA,
        pltpu.SemaphoreType.DMA,
        pltpu.SemaphoreType.DMA,
    ],
    compiler_params=pltpu.CompilerParams(use_tc_tiling_on_sc=False),
)
def _embed_sc(tok_hbm, table_hbm, out_hbm, idx_all, rows0, rows1, g0, g1, w0, w1):
    wid = lax.axis_index("s") * NC + lax.axis_index("c")
    base = wid * PER_W
    rows = (rows0, rows1)
    gsem = (g0, g1)
    wsem = (w0, w1)

    pltpu.sync_copy(tok_hbm.at[wid], idx_all)
    pltpu.async_copy(table_hbm.at[idx_all.at[0]], rows[0], gsem[0])

    for c in range(NCHUNK):
        b = c & 1
        if c + 1 < NCHUNK:
            if c >= 1:
                pltpu.make_async_copy(
                    rows[1 - b],
                    out_hbm.at[pl.ds(base + (c - 1) * CHUNK, CHUNK)],
                    wsem[1 - b],
                ).wait()
            pltpu.async_copy(table_hbm.at[idx_all.at[c + 1]], rows[1 - b], gsem[1 - b])
        pltpu.make_async_copy(table_hbm.at[idx_all.at[c]], rows[b], gsem[b]).wait()

        def scale_row(j, carry, _rows=rows[b]):
            lo = _rows[j, pl.ds(0, 16)] * SCALE
            hi = _rows[j, pl.ds(16, 16)] * SCALE
            _rows[j, pl.ds(0, 16)] = lo
            _rows[j, pl.ds(16, 16)] = hi
            return carry

        lax.fori_loop(0, CHUNK, scale_row, 0, unroll=8)
        pltpu.async_copy(rows[b], out_hbm.at[pl.ds(base + c * CHUNK, CHUNK)], wsem[b])

    pltpu.make_async_copy(
        rows[0], out_hbm.at[pl.ds(base + (NCHUNK - 2) * CHUNK, CHUNK)], wsem[0]
    ).wait()
    pltpu.make_async_copy(
        rows[1], out_hbm.at[pl.ds(base + (NCHUNK - 1) * CHUNK, CHUNK)], wsem[1]
    ).wait()


def kernel(tokens, embedding_weight):
    toks = tokens.reshape(NW, NCHUNK, CHUNK)
    out = _embed_sc(toks, embedding_weight)
    return out.reshape(BATCH, SEQ, EMB)
